# baseline (speedup 1.0000x reference)
import functools

import jax
import jax.numpy as jnp
from jax import lax
from jax.experimental import pallas as pl
from jax.experimental.pallas import tpu as pltpu
from jax.experimental.pallas import tpu_sc as plsc

_SCALE = 3.1622776601683795

_NUM_WORKERS = 32
_CHUNK = 128


def _emb_call(H, D, B):
    mesh = plsc.VectorSubcoreMesh(core_axis_name="c", subcore_axis_name="s")
    n_tasks = H

    @functools.partial(
        pl.kernel,
        mesh=mesh,
        out_type=jax.ShapeDtypeStruct((H, D // 8, B // _CHUNK, 8, _CHUNK),
                                      jnp.float32),
        scratch_types=(
            [pltpu.VMEM((n_tasks, _CHUNK), jnp.int32)]
            + [pltpu.VMEM((_CHUNK, D), jnp.float32) for _ in range(4)]
            + [pltpu.VMEM((D // 8, 8, _CHUNK), jnp.float32) for _ in range(4)]
            + [pltpu.SemaphoreType.DMA for _ in range(8)]
        ),
        compiler_params=pltpu.CompilerParams(use_tc_tiling_on_sc=False,
                                             needs_layout_passes=False),
    )
    def emb(xt_hbm, table_hbm, out_hbm, idx_v,
            b0, b1, b2, b3, t0, t1, t2, t3, g0, g1, g2, g3, s0, s1, s2, s3):
        bufs = (b0, b1, b2, b3)
        tbufs = (t0, t1, t2, t3)
        gs = (g0, g1, g2, g3)
        ss = (s0, s1, s2, s3)
        wid = lax.axis_index("s") * 2 + lax.axis_index("c")
        pltpu.sync_copy(xt_hbm.at[:, pl.ds(wid * _CHUNK, _CHUNK)], idx_v)

        iota = lax.iota(jnp.int32, 16)
        idx_b = [iota + g * 16 for g in range(_CHUNK // 16)]

        def gather_start(j, k):
            pltpu.async_copy(table_hbm.at[idx_v.at[j]], bufs[k], gs[k])

        def gather_wait(j, k):
            pltpu.make_async_copy(table_hbm.at[idx_v.at[j]], bufs[k], gs[k]).wait()

        def store_start(j, k):
            pltpu.async_copy(tbufs[k], out_hbm.at[j, :, wid], ss[k])

        def store_wait(k):
            pltpu.make_async_copy(tbufs[k], out_hbm.at[0, :, wid], ss[k]).wait()

        def transpose_scale(k):
            buf, tbuf = bufs[k], tbufs[k]

            def body_r(r, carry):
                for s in range(8):
                    d = r * 8 + s
                    idx_d = jnp.full((16,), 0, jnp.int32) + d
                    for g in range(_CHUNK // 16):
                        vals = plsc.load_gather(buf, [idx_b[g], idx_d])
                        tbuf[r, s, pl.ds(g * 16, 16)] = vals * _SCALE
                return carry

            lax.fori_loop(0, D // 8, body_r, 0)

        gather_start(0, 0)
        gather_start(1, 1)
        gather_start(2, 2)
        gather_wait(0, 0)
        transpose_scale(0)
        store_start(0, 0)
        gather_start(3, 3)
        gather_wait(1, 1)
        transpose_scale(1)
        store_start(1, 1)

        def step(jj, carry):
            j0 = 2 + jj * 4
            for t in range(4):
                j = j0 + t
                k = (2 + t) % 4
                kg = t % 4
                store_wait(kg)
                gather_start(j + 2, kg)
                gather_wait(j, k)
                transpose_scale(k)
                store_start(j, k)
            return carry

        lax.fori_loop(0, (n_tasks - 4) // 4, step, 0)

        gather_wait(n_tasks - 2, 2)
        transpose_scale(2)
        store_start(n_tasks - 2, 2)
        gather_wait(n_tasks - 1, 3)
        transpose_scale(3)
        store_start(n_tasks - 1, 3)
        for k in range(4):
            store_wait(k)

    return emb


def kernel(x, table):
    B, H = x.shape
    V, D = table.shape
    assert B % (_NUM_WORKERS * _CHUNK) == 0 or B == _NUM_WORKERS * _CHUNK
    assert D % 16 == 0 and H % 4 == 0 and H >= 8
    xt = jnp.transpose(x.astype(jnp.int32))
    out5 = _emb_call(H, D, B)(xt, table)
    out = jnp.transpose(
        jnp.reshape(jnp.transpose(out5, (0, 1, 3, 2, 4)), (H, D, B)),
        (2, 0, 1))
    return out


# speedup vs baseline: 1.1505x; 1.1505x over previous
import functools

import jax
import jax.numpy as jnp
from jax import lax
from jax.experimental import pallas as pl
from jax.experimental.pallas import tpu as pltpu
from jax.experimental.pallas import tpu_sc as plsc

_SCALE = 3.1622776601683795

_NUM_WORKERS = 32
_CHUNK = 128


def _emb_call(H, D, B):
    mesh = plsc.VectorSubcoreMesh(core_axis_name="c", subcore_axis_name="s")
    n_tasks = H

    @functools.partial(
        pl.kernel,
        mesh=mesh,
        out_type=jax.ShapeDtypeStruct((H, D // 8, B // _CHUNK, 8, _CHUNK),
                                      jnp.float32),
        scratch_types=(
            [pltpu.VMEM((n_tasks, _CHUNK), jnp.int32),
             pltpu.VMEM((4, _CHUNK), jnp.int32)]
            + [pltpu.VMEM((_CHUNK, 2 * D), jnp.float32) for _ in range(4)]
            + [pltpu.VMEM((D // 8, 8, _CHUNK), jnp.float32) for _ in range(4)]
            + [pltpu.SemaphoreType.DMA for _ in range(8)]
        ),
        compiler_params=pltpu.CompilerParams(use_tc_tiling_on_sc=True,
                                             needs_layout_passes=False),
    )
    def emb(xt_hbm, table_hbm, out_hbm, idx_v, hidx,
            b0, b1, b2, b3, t0, t1, t2, t3, g0, g1, g2, g3, s0, s1, s2, s3):
        bufs = (b0, b1, b2, b3)
        tbufs = (t0, t1, t2, t3)
        gs = (g0, g1, g2, g3)
        ss = (s0, s1, s2, s3)
        wid = lax.axis_index("s") * 2 + lax.axis_index("c")
        pltpu.sync_copy(xt_hbm.at[:, pl.ds(wid * _CHUNK, _CHUNK)], idx_v)

        iota = lax.iota(jnp.int32, 16)
        idx_d8 = [(iota >> 3) + 2 * c for c in range(D // 16)]
        idx_dm = iota & 7

        def gather_start(j, k):
            for g in range(_CHUNK // 16):
                hidx[k, pl.ds(g * 16, 16)] = (
                    idx_v[j, pl.ds(g * 16, 16)] >> 1)
            pltpu.async_copy(table_hbm.at[hidx.at[k]], bufs[k], gs[k])

        def gather_wait(k):
            pltpu.make_async_copy(table_hbm.at[hidx.at[k]], bufs[k], gs[k]).wait()

        def store_start(j, k):
            pltpu.async_copy(tbufs[k], out_hbm.at[j, :, wid], ss[k])

        def store_wait(k):
            pltpu.make_async_copy(tbufs[k], out_hbm.at[0, :, wid], ss[k]).wait()

        def transpose_scale(j, k):
            buf, tbuf = bufs[k], tbufs[k]

            def body_g(g, carry):
                b0 = g * 16
                voffs = (idx_v[j, pl.ds(b0, 16)] & 1) * D
                for l in range(16):
                    b = b0 + l
                    voff = voffs[l]
                    bfull = jnp.full((16,), 0, jnp.int32) + b
                    for c in range(D // 16):
                        val = buf[b, pl.ds(voff + c * 16, 16)] * _SCALE
                        plsc.store_scatter(tbuf, [idx_d8[c], idx_dm, bfull], val)
                return carry

            lax.fori_loop(0, _CHUNK // 16, body_g, 0)

        gather_start(0, 0)
        gather_start(1, 1)
        gather_start(2, 2)
        gather_wait(0)
        transpose_scale(0, 0)
        store_start(0, 0)
        gather_start(3, 3)
        gather_wait(1)
        transpose_scale(1, 1)
        store_start(1, 1)

        def step(jj, carry):
            j0 = 2 + jj * 4
            for t in range(4):
                j = j0 + t
                k = (2 + t) % 4
                kg = t % 4
                store_wait(kg)
                gather_start(j + 2, kg)
                gather_wait(k)
                transpose_scale(j, k)
                store_start(j, k)
            return carry

        lax.fori_loop(0, (n_tasks - 4) // 4, step, 0)

        gather_wait(2)
        transpose_scale(n_tasks - 2, 2)
        store_start(n_tasks - 2, 2)
        gather_wait(3)
        transpose_scale(n_tasks - 1, 3)
        store_start(n_tasks - 1, 3)
        for k in range(4):
            store_wait(k)

    return emb


def kernel(x, table):
    B, H = x.shape
    V, D = table.shape
    assert B == _NUM_WORKERS * _CHUNK
    assert D % 16 == 0 and H % 4 == 0 and H >= 8 and V % 2 == 0
    xt = jnp.transpose(x.astype(jnp.int32))
    t2 = jnp.reshape(table, (V // 2, 2 * D))
    out5 = _emb_call(H, D, B)(xt, t2)
    out = jnp.transpose(
        jnp.reshape(jnp.transpose(out5, (0, 1, 3, 2, 4)), (H, D, B)),
        (2, 0, 1))
    return out
